# baseline (speedup 1.0000x reference)
import numpy as np
import jax
import jax.numpy as jnp
from jax.experimental import pallas as pl
from jax.experimental.pallas import tpu as pltpu

_FEAT_STRIDE = 16
_RPN_BATCHSIZE = 256
_NUM_FG = 128
_POS_OVERLAP = 0.7
_NEG_OVERLAP = 0.3
_B = 4
_P = 2
_A = 9
_H = 64
_W = 64
_K = _H * _W
_N = _K * _A
_BR = 288
_RS = _P * _BR
_RW = 128
_KR = 32
_G = 20


def _np_whctrs(a):
    w = a[2] - a[0] + 1.0
    h = a[3] - a[1] + 1.0
    return w, h, a[0] + 0.5 * (w - 1.0), a[1] + 0.5 * (h - 1.0)


def _np_mkanchors(ws, hs, xc, yc):
    ws = np.asarray(ws, dtype=np.float64).reshape(-1, 1)
    hs = np.asarray(hs, dtype=np.float64).reshape(-1, 1)
    return np.hstack((xc - 0.5 * (ws - 1.0), yc - 0.5 * (hs - 1.0),
                      xc + 0.5 * (ws - 1.0), yc + 0.5 * (hs - 1.0)))


def _np_gen_anchors(base_size=16, ratios=(0.5, 1.0, 2.0), scales=(8.0, 16.0, 32.0)):
    ratios = np.array(ratios)
    scales = np.array(scales)
    base = np.array([1.0, 1.0, base_size, base_size]) - 1.0
    w, h, xc, yc = _np_whctrs(base)
    size = w * h
    ws = np.round(np.sqrt(size / ratios))
    hs = np.round(ws * ratios)
    ra = _np_mkanchors(ws, hs, xc, yc)
    out = []
    for i in range(ra.shape[0]):
        w, h, xc, yc = _np_whctrs(ra[i])
        out.append(_np_mkanchors(w * scales, h * scales, xc, yc))
    return np.vstack(out).astype(np.float32)


def _np_all_anchors_m():
    anch = _np_gen_anchors()
    sx, sy = np.meshgrid(np.arange(_W) * _FEAT_STRIDE, np.arange(_H) * _FEAT_STRIDE)
    shifts = np.stack([sx.ravel(), sy.ravel(), sx.ravel(), sy.ravel()], axis=1).astype(np.float32)
    alla = anch[None, :, :] + shifts[:, None, :]
    allm = np.ascontiguousarray(np.transpose(alla, (1, 0, 2)))
    return allm.reshape(_N, 4)


_ALL_ANCHORS = _np_all_anchors_m()
_AX1 = np.tile(_ALL_ANCHORS[:, 0].reshape(_BR, _RW), (_P, 1))
_AY1 = np.tile(_ALL_ANCHORS[:, 1].reshape(_BR, _RW), (_P, 1))
_AX2 = np.tile(_ALL_ANCHORS[:, 2].reshape(_BR, _RW), (_P, 1))
_AY2 = np.tile(_ALL_ANCHORS[:, 3].reshape(_BR, _RW), (_P, 1))


def _atl_body(fgv_ref, ax1_ref, ay1_ref, ax2_ref, ay2_ref,
              gx1_ref, gy1_ref, gx2_ref, gy2_ref, im_ref,
              lab_ref, tgt_ref, inw_ref, outw_ref):
    f32 = jnp.float32
    ax1 = ax1_ref[...]
    ay1 = ay1_ref[...]
    ax2 = ax2_ref[...]
    ay2 = ay2_ref[...]
    aw = ax2 - ax1 + 1.0
    ah = ay2 - ay1 + 1.0
    aarea = aw * ah
    ecx = ax1 + 0.5 * aw
    ecy = ay1 + 0.5 * ah
    im_h = im_ref[0, 0]
    im_w = im_ref[0, 1]
    ins = (ax1 >= 0.0) & (ay1 >= 0.0) & (ax2 < im_w) & (ay2 < im_h)
    scores = jnp.concatenate([fgv_ref[p, _BR:2 * _BR, :] for p in range(_P)], axis=0)

    si = jax.lax.broadcasted_iota(jnp.int32, (_RS, _RS), 0)
    sj = jax.lax.broadcasted_iota(jnp.int32, (_RS, _RS), 1)
    smat = (si // _BR == sj // _BR).astype(f32)

    def seg_sum(x):
        rs = jnp.sum(x, axis=1, keepdims=True)
        return jnp.dot(smat, rs, preferred_element_type=f32)

    rowb = jax.lax.broadcasted_iota(jnp.int32, (_RS, 1), 0) // _BR

    def batch_rows(gref):
        gt = gref[0]
        out = jnp.broadcast_to(gt[0:1, :], (_RS, _G))
        for p in range(1, _P):
            out = jnp.where(rowb == p, gt[p:p + 1, :], out)
        return out

    gx1m = batch_rows(gx1_ref)
    gy1m = batch_rows(gy1_ref)
    gx2m = batch_rows(gx2_ref)
    gy2m = batch_rows(gy2_ref)

    cur_max = jnp.full((_RS, _RW), -jnp.inf, dtype=f32)
    bx1 = jnp.zeros((_RS, _RW), dtype=f32)
    by1 = jnp.zeros((_RS, _RW), dtype=f32)
    bx2 = jnp.zeros((_RS, _RW), dtype=f32)
    by2 = jnp.zeros((_RS, _RW), dtype=f32)
    eqpack = jnp.zeros((_RS, _RW), dtype=jnp.int32)
    flagbits = jnp.zeros((_RS, 1), dtype=jnp.int32)
    for g in range(_G):
        gx1 = gx1m[:, g:g + 1]
        gy1 = gy1m[:, g:g + 1]
        gx2 = gx2m[:, g:g + 1]
        gy2 = gy2m[:, g:g + 1]
        gw = gx2 - gx1 + 1.0
        gh = gy2 - gy1 + 1.0
        garea = gw * gh
        ix1 = jnp.maximum(ax1, gx1)
        iy1 = jnp.maximum(ay1, gy1)
        ix2 = jnp.minimum(ax2, gx2)
        iy2 = jnp.minimum(ay2, gy2)
        inter = jnp.maximum(ix2 - ix1 + 1.0, 0.0) * jnp.maximum(iy2 - iy1 + 1.0, 0.0)
        ov = inter / (aarea + garea - inter)
        gtz = (gw == 1.0) & (gh == 1.0)
        ov = jnp.where(gtz, jnp.zeros_like(ov), ov)
        ov = jnp.where(ins, ov, -1.0)
        rmg = jnp.max(ov, axis=1, keepdims=True)
        eqpack = eqpack | jnp.where(ov == rmg, jnp.int32(1 << g), 0)
        bparts = [jnp.max(rmg[p * _BR:(p + 1) * _BR, :], axis=0, keepdims=True)
                  for p in range(_P)]
        bmax = jnp.broadcast_to(bparts[0], (_RS, 1))
        for p in range(1, _P):
            bmax = jnp.where(rowb == p, bparts[p], bmax)
        bmax = jnp.where(bmax == 0.0, 1e-5, bmax)
        flagbits = flagbits | jnp.where(rmg == bmax, jnp.int32(1 << g), 0)
        upd = ov > cur_max
        cur_max = jnp.where(upd, ov, cur_max)
        bx1 = jnp.where(upd, gx1, bx1)
        by1 = jnp.where(upd, gy1, by1)
        bx2 = jnp.where(upd, gx2, bx2)
        by2 = jnp.where(upd, gy2, by2)

    bw_ = bx2 - bx1 + 1.0
    bh_ = by2 - by1 + 1.0
    bcx = bx1 + 0.5 * bw_
    bcy = by1 + 0.5 * bh_
    dx = (bcx - ecx) / aw
    dy = (bcy - ecy) / ah
    dwv = jnp.log(bw_ / aw)
    dhv = jnp.log(bh_ / ah)
    zeros = jnp.zeros((_RS, _RW), dtype=f32)
    dx = jnp.where(ins, dx, zeros)
    dy = jnp.where(ins, dy, zeros)
    dwv = jnp.where(ins, dwv, zeros)
    dhv = jnp.where(ins, dhv, zeros)
    comps = (dx, dy, dwv, dhv)
    for p in range(_P):
        for a in range(_A):
            s = p * _BR + a * _KR
            for d in range(4):
                o = p * 4 * _BR + (4 * a + d) * _KR
                tgt_ref[o:o + _KR, :] = comps[d][s:s + _KR, :]

    labels = jnp.where(cur_max < _NEG_OVERLAP, 0.0, -1.0)
    keep = (eqpack & flagbits) != 0
    labels = jnp.where(keep, 1.0, labels)
    labels = jnp.where(cur_max >= _POS_OVERLAP, 1.0, labels)
    labels = jnp.where(ins, labels, -1.0)

    li = jax.lax.broadcasted_iota(jnp.int32, (_RW, _RW), 0)
    lj = jax.lax.broadcasted_iota(jnp.int32, (_RW, _RW), 1)
    m128 = (li <= lj).astype(f32)
    ki = jax.lax.broadcasted_iota(jnp.int32, (_P * _KR, _P * _KR), 0)
    kj = jax.lax.broadcasted_iota(jnp.int32, (_P * _KR, _P * _KR), 1)
    lbd = ((kj < ki) & (ki // _KR == kj // _KR)).astype(f32)

    def n_rank_incl(maskf):
        perk2 = jnp.concatenate(
            [sum(maskf[p * _BR + a * _KR: p * _BR + (a + 1) * _KR, :]
                 for a in range(_A)) for p in range(_P)], axis=0)
        rowc = jnp.dot(perk2, m128, preferred_element_type=f32)
        rtot = rowc[:, _RW - 1:_RW]
        rex = jnp.dot(lbd, rtot, preferred_element_type=f32)
        exk = rowc - perk2 + rex
        blocks = []
        for p in range(_P):
            acc = exk[p * _KR:(p + 1) * _KR, :]
            for a in range(_A):
                acc = acc + maskf[p * _BR + a * _KR: p * _BR + (a + 1) * _KR, :]
                blocks.append(acc)
        return jnp.concatenate(blocks, axis=0)

    fg = labels == 1.0
    fgf = jnp.where(fg, 1.0, 0.0)
    sum_fg = seg_sum(fgf)
    excess_fg = jnp.maximum(sum_fg - float(_NUM_FG), 0.0)
    fgrank = n_rank_incl(fgf) - 1.0
    labels = jnp.where(fg & (fgrank < excess_fg), -1.0, labels)

    bg = labels == 0.0
    bgf = jnp.where(bg, 1.0, 0.0)
    sum_bg = seg_sum(bgf)
    num_bg = float(_RPN_BATCHSIZE) - sum_fg
    excess_bg = jnp.maximum(sum_bg - num_bg, 0.0)
    keys = jax.lax.bitcast_convert_type(scores, jnp.int32)
    khi = jax.lax.shift_right_logical(keys, 15).astype(f32)
    klo = (keys & 0x7FFF).astype(f32)
    thr_hi = jnp.zeros((_RS, 1), dtype=f32)
    thr_lo = jnp.zeros((_RS, 1), dtype=f32)
    for bit in range(30, -1, -1):
        if bit >= 15:
            cand_hi = thr_hi + float(1 << (bit - 15))
            cand_lo = thr_lo
        else:
            cand_hi = thr_hi
            cand_lo = thr_lo + float(1 << bit)
        ge = (khi > cand_hi) | ((khi == cand_hi) & (klo >= cand_lo))
        cnt = seg_sum(jnp.where(bg & ge, 1.0, 0.0))
        take = cnt >= excess_bg
        thr_hi = jnp.where(take, cand_hi, thr_hi)
        thr_lo = jnp.where(take, cand_lo, thr_lo)
    kgt = (khi > thr_hi) | ((khi == thr_hi) & (klo > thr_lo))
    keq = (khi == thr_hi) & (klo == thr_lo)
    ngt = seg_sum(jnp.where(bg & kgt, 1.0, 0.0))
    rtie = excess_bg - ngt
    tie = bg & keq
    tierank = n_rank_incl(jnp.where(tie, 1.0, 0.0)) - 1.0
    disable = (bg & kgt) | (tie & (tierank < rtie))
    labels = jnp.where(disable, -1.0, labels)

    num_ex = seg_sum(jnp.where(labels >= 0.0, 1.0, 0.0))
    inv = 1.0 / num_ex
    inw = jnp.where(labels == 1.0, 1.0, 0.0)
    outw = jnp.where(labels >= 0.0, inv, jnp.zeros((_RS, _RW), dtype=f32))

    lab_ref[...] = labels
    for p in range(_P):
        for a in range(_A):
            s = p * _BR + a * _KR
            blk_in = inw[s:s + _KR, :]
            blk_out = outw[s:s + _KR, :]
            for d in range(4):
                o = p * 4 * _BR + (4 * a + d) * _KR
                inw_ref[o:o + _KR, :] = blk_in
                outw_ref[o:o + _KR, :] = blk_out


def kernel(rpn_cls_score, gt_boxes, im_info, num_boxes, fg_prob):
    B = gt_boxes.shape[0]
    H, W, A = _H, _W, _A
    fgv = fg_prob.reshape(B, 2 * _BR, _RW)
    nsteps = _B // _P
    gx1 = gt_boxes[:, :, 0].reshape(nsteps, _P, _G)
    gy1 = gt_boxes[:, :, 1].reshape(nsteps, _P, _G)
    gx2 = gt_boxes[:, :, 2].reshape(nsteps, _P, _G)
    gy2 = gt_boxes[:, :, 3].reshape(nsteps, _P, _G)
    labels, tgt, inw, outw = pl.pallas_call(
        _atl_body,
        grid=(nsteps,),
        in_specs=[pl.BlockSpec((_P, 2 * _BR, _RW), lambda p: (p, 0, 0)),
                  pl.BlockSpec((_RS, _RW), lambda p: (0, 0)),
                  pl.BlockSpec((_RS, _RW), lambda p: (0, 0)),
                  pl.BlockSpec((_RS, _RW), lambda p: (0, 0)),
                  pl.BlockSpec((_RS, _RW), lambda p: (0, 0)),
                  pl.BlockSpec((1, _P, _G), lambda p: (p, 0, 0)),
                  pl.BlockSpec((1, _P, _G), lambda p: (p, 0, 0)),
                  pl.BlockSpec((1, _P, _G), lambda p: (p, 0, 0)),
                  pl.BlockSpec((1, _P, _G), lambda p: (p, 0, 0)),
                  pl.BlockSpec(memory_space=pltpu.SMEM)],
        out_specs=[pl.BlockSpec((_RS, _RW), lambda p: (p, 0)),
                   pl.BlockSpec((4 * _RS, _RW), lambda p: (p, 0)),
                   pl.BlockSpec((4 * _RS, _RW), lambda p: (p, 0)),
                   pl.BlockSpec((4 * _RS, _RW), lambda p: (p, 0))],
        out_shape=[jax.ShapeDtypeStruct((_B * _BR, _RW), jnp.float32),
                   jax.ShapeDtypeStruct((4 * _B * _BR, _RW), jnp.float32),
                   jax.ShapeDtypeStruct((4 * _B * _BR, _RW), jnp.float32),
                   jax.ShapeDtypeStruct((4 * _B * _BR, _RW), jnp.float32)],
    )(fgv, jnp.asarray(_AX1), jnp.asarray(_AY1), jnp.asarray(_AX2),
      jnp.asarray(_AY2), gx1, gy1, gx2, gy2, im_info)

    labels_out = labels.reshape(B, 1, A * H, W)
    targets_out = tgt.reshape(B, 4 * A, H, W)
    inw_out = inw.reshape(B, 4 * A, H, W)
    outw_out = outw.reshape(B, 4 * A, H, W)
    return labels_out, targets_out, inw_out, outw_out


# speedup vs baseline: 1.5483x; 1.5483x over previous
import numpy as np
import jax
import jax.numpy as jnp
from jax.experimental import pallas as pl
from jax.experimental.pallas import tpu as pltpu

_FEAT_STRIDE = 16
_RPN_BATCHSIZE = 256
_NUM_FG = 128
_POS_OVERLAP = 0.7
_NEG_OVERLAP = 0.3
_A = 9
_H = 64
_W = 64
_K = _H * _W
_N = _K * _A
_RH = 288
_RW = 128
_KR = 32
_G = 20


def _np_whctrs(a):
    w = a[2] - a[0] + 1.0
    h = a[3] - a[1] + 1.0
    return w, h, a[0] + 0.5 * (w - 1.0), a[1] + 0.5 * (h - 1.0)


def _np_mkanchors(ws, hs, xc, yc):
    ws = np.asarray(ws, dtype=np.float64).reshape(-1, 1)
    hs = np.asarray(hs, dtype=np.float64).reshape(-1, 1)
    return np.hstack((xc - 0.5 * (ws - 1.0), yc - 0.5 * (hs - 1.0),
                      xc + 0.5 * (ws - 1.0), yc + 0.5 * (hs - 1.0)))


def _np_gen_anchors(base_size=16, ratios=(0.5, 1.0, 2.0), scales=(8.0, 16.0, 32.0)):
    ratios = np.array(ratios)
    scales = np.array(scales)
    base = np.array([1.0, 1.0, base_size, base_size]) - 1.0
    w, h, xc, yc = _np_whctrs(base)
    size = w * h
    ws = np.round(np.sqrt(size / ratios))
    hs = np.round(ws * ratios)
    ra = _np_mkanchors(ws, hs, xc, yc)
    out = []
    for i in range(ra.shape[0]):
        w, h, xc, yc = _np_whctrs(ra[i])
        out.append(_np_mkanchors(w * scales, h * scales, xc, yc))
    return np.vstack(out).astype(np.float32)


def _np_all_anchors_m():
    anch = _np_gen_anchors()
    sx, sy = np.meshgrid(np.arange(_W) * _FEAT_STRIDE, np.arange(_H) * _FEAT_STRIDE)
    shifts = np.stack([sx.ravel(), sy.ravel(), sx.ravel(), sy.ravel()], axis=1).astype(np.float32)
    alla = anch[None, :, :] + shifts[:, None, :]
    allm = np.ascontiguousarray(np.transpose(alla, (1, 0, 2)))
    return allm.reshape(_N, 4)


_ALL_ANCHORS = _np_all_anchors_m()
_AX1 = _ALL_ANCHORS[:, 0].reshape(_RH, _RW)
_AY1 = _ALL_ANCHORS[:, 1].reshape(_RH, _RW)
_AX2 = _ALL_ANCHORS[:, 2].reshape(_RH, _RW)
_AY2 = _ALL_ANCHORS[:, 3].reshape(_RH, _RW)


def _atl_body(scores_ref, ax1_ref, ay1_ref, ax2_ref, ay2_ref,
              gx1_ref, gy1_ref, gx2_ref, gy2_ref, im_ref,
              lab_ref, tgt_ref, inw_ref, outw_ref):
    b = pl.program_id(0)
    ax1 = ax1_ref[...]
    ay1 = ay1_ref[...]
    ax2 = ax2_ref[...]
    ay2 = ay2_ref[...]
    aw = ax2 - ax1 + 1.0
    ah = ay2 - ay1 + 1.0
    aarea = aw * ah
    ecx = ax1 + 0.5 * aw
    ecy = ay1 + 0.5 * ah
    im_h = im_ref[0, 0]
    im_w = im_ref[0, 1]
    ins = (ax1 >= 0.0) & (ay1 >= 0.0) & (ax2 < im_w) & (ay2 < im_h)
    scores = scores_ref[0]

    cur_max = jnp.full((_RH, _RW), -jnp.inf, dtype=jnp.float32)
    bx1 = jnp.zeros((_RH, _RW), dtype=jnp.float32)
    by1 = jnp.zeros((_RH, _RW), dtype=jnp.float32)
    bx2 = jnp.zeros((_RH, _RW), dtype=jnp.float32)
    by2 = jnp.zeros((_RH, _RW), dtype=jnp.float32)
    eqpack = jnp.zeros((_RH, _RW), dtype=jnp.int32)
    flagbits = jnp.zeros((_RH, 1), dtype=jnp.int32)
    for g in range(_G):
        gx1 = gx1_ref[b, g]
        gy1 = gy1_ref[b, g]
        gx2 = gx2_ref[b, g]
        gy2 = gy2_ref[b, g]
        gw = gx2 - gx1 + 1.0
        gh = gy2 - gy1 + 1.0
        garea = gw * gh
        ix1 = jnp.maximum(ax1, gx1)
        iy1 = jnp.maximum(ay1, gy1)
        ix2 = jnp.minimum(ax2, gx2)
        iy2 = jnp.minimum(ay2, gy2)
        inter = jnp.maximum(ix2 - ix1 + 1.0, 0.0) * jnp.maximum(iy2 - iy1 + 1.0, 0.0)
        ov = inter / (aarea + garea - inter)
        gtz = (gw == 1.0) & (gh == 1.0)
        ov = jnp.where(gtz, jnp.zeros_like(ov), ov)
        ov = jnp.where(ins, ov, -1.0)
        rmg = jnp.max(ov, axis=1, keepdims=True)
        gm = jnp.max(rmg)
        gm = jnp.where(gm == 0.0, 1e-5, gm)
        eqpack = eqpack | jnp.where(ov == rmg, jnp.int32(1 << g), 0)
        flagbits = flagbits | jnp.where(rmg == gm, jnp.int32(1 << g), 0)
        upd = ov > cur_max
        cur_max = jnp.where(upd, ov, cur_max)
        bx1 = jnp.where(upd, gx1, bx1)
        by1 = jnp.where(upd, gy1, by1)
        bx2 = jnp.where(upd, gx2, bx2)
        by2 = jnp.where(upd, gy2, by2)

    labels = jnp.where(cur_max < _NEG_OVERLAP, 0.0, -1.0)
    keep = (eqpack & flagbits) != 0
    labels = jnp.where(keep, 1.0, labels)
    labels = jnp.where(cur_max >= _POS_OVERLAP, 1.0, labels)
    labels = jnp.where(ins, labels, -1.0)

    li = jax.lax.broadcasted_iota(jnp.int32, (_RW, _RW), 0)
    lj = jax.lax.broadcasted_iota(jnp.int32, (_RW, _RW), 1)
    m128 = (li <= lj).astype(jnp.float32)
    ri = jax.lax.broadcasted_iota(jnp.int32, (_KR, _KR), 0)
    rj = jax.lax.broadcasted_iota(jnp.int32, (_KR, _KR), 1)
    l32 = (rj < ri).astype(jnp.float32)

    def n_rank_incl(maskf):
        m3 = maskf.reshape(_A, _KR, _RW)
        perk = jnp.sum(m3, axis=0)
        rowc = jnp.dot(perk, m128, preferred_element_type=jnp.float32)
        rtot = rowc[:, _RW - 1:_RW]
        rex = jnp.dot(l32, rtot, preferred_element_type=jnp.float32)
        exk = rowc - perk + rex
        acc = exk + m3[0]
        parts = [acc]
        for a in range(1, _A):
            acc = acc + m3[a]
            parts.append(acc)
        incl = jnp.concatenate([p[None] for p in parts], axis=0)
        return incl.reshape(_RH, _RW)

    fg = labels == 1.0
    fgf = jnp.where(fg, 1.0, 0.0)
    sum_fg = jnp.sum(fgf)
    excess_fg = jnp.maximum(sum_fg - float(_NUM_FG), 0.0)
    fgrank = n_rank_incl(fgf) - 1.0
    labels = jnp.where(fg & (fgrank < excess_fg), -1.0, labels)

    bg = labels == 0.0
    bgf = jnp.where(bg, 1.0, 0.0)
    sum_bg = jnp.sum(bgf)
    num_bg = float(_RPN_BATCHSIZE) - sum_fg
    excess_bg = jnp.maximum(sum_bg - num_bg, 0.0)
    keys = jax.lax.bitcast_convert_type(scores, jnp.int32)
    thr = jnp.int32(0)
    for bit in range(30, -1, -1):
        cand = thr | jnp.int32(1 << bit)
        cnt = jnp.sum(jnp.where(bg & (keys >= cand), 1.0, 0.0))
        thr = jnp.where(cnt >= excess_bg, cand, thr)
    ngt = jnp.sum(jnp.where(bg & (keys > thr), 1.0, 0.0))
    rtie = excess_bg - ngt
    tie = bg & (keys == thr)
    tierank = n_rank_incl(jnp.where(tie, 1.0, 0.0)) - 1.0
    disable = (bg & (keys > thr)) | (tie & (tierank < rtie))
    labels = jnp.where(disable, -1.0, labels)

    bw_ = bx2 - bx1 + 1.0
    bh_ = by2 - by1 + 1.0
    bcx = bx1 + 0.5 * bw_
    bcy = by1 + 0.5 * bh_
    dx = (bcx - ecx) / aw
    dy = (bcy - ecy) / ah
    dwv = jnp.log(bw_ / aw)
    dhv = jnp.log(bh_ / ah)
    zeros = jnp.zeros((_RH, _RW), dtype=jnp.float32)
    dx = jnp.where(ins, dx, zeros)
    dy = jnp.where(ins, dy, zeros)
    dwv = jnp.where(ins, dwv, zeros)
    dhv = jnp.where(ins, dhv, zeros)

    num_ex = jnp.sum(jnp.where(labels >= 0.0, 1.0, 0.0))
    inv = 1.0 / num_ex
    inw = jnp.where(labels == 1.0, 1.0, 0.0)
    outw = jnp.where(labels >= 0.0, inv, 0.0)

    lab_ref[0] = labels
    comps = (dx, dy, dwv, dhv)
    for a in range(_A):
        s = a * _KR
        blk_in = inw[s:s + _KR, :]
        blk_out = outw[s:s + _KR, :]
        for d in range(4):
            o = (4 * a + d) * _KR
            tgt_ref[0, o:o + _KR, :] = comps[d][s:s + _KR, :]
            inw_ref[0, o:o + _KR, :] = blk_in
            outw_ref[0, o:o + _KR, :] = blk_out


def kernel(rpn_cls_score, gt_boxes, im_info, num_boxes, fg_prob):
    B = gt_boxes.shape[0]
    H, W, A = _H, _W, _A
    fgv = fg_prob.reshape(B, 2 * A * _K // _RW, _RW)
    gx1 = gt_boxes[:, :, 0]
    gy1 = gt_boxes[:, :, 1]
    gx2 = gt_boxes[:, :, 2]
    gy2 = gt_boxes[:, :, 3]

    score_spec = pl.BlockSpec((1, _RH, _RW), lambda b: (b, 1, 0))
    vspec_b = pl.BlockSpec((1, _RH, _RW), lambda b: (b, 0, 0))
    vspec_b4 = pl.BlockSpec((1, 4 * _RH, _RW), lambda b: (b, 0, 0))
    vspec_c = pl.BlockSpec((_RH, _RW), lambda b: (0, 0))
    sspec = pl.BlockSpec(memory_space=pltpu.SMEM)

    labels, tgt, inw, outw = pl.pallas_call(
        _atl_body,
        grid=(B,),
        in_specs=[score_spec, vspec_c, vspec_c, vspec_c, vspec_c,
                  sspec, sspec, sspec, sspec, sspec],
        out_specs=[vspec_b, vspec_b4, vspec_b4, vspec_b4],
        out_shape=[jax.ShapeDtypeStruct((B, _RH, _RW), jnp.float32),
                   jax.ShapeDtypeStruct((B, 4 * _RH, _RW), jnp.float32),
                   jax.ShapeDtypeStruct((B, 4 * _RH, _RW), jnp.float32),
                   jax.ShapeDtypeStruct((B, 4 * _RH, _RW), jnp.float32)],
    )(fgv, jnp.asarray(_AX1), jnp.asarray(_AY1), jnp.asarray(_AX2),
      jnp.asarray(_AY2), gx1, gy1, gx2, gy2, im_info)

    labels_out = labels.reshape(B, 1, A * H, W)
    targets_out = tgt.reshape(B, 4 * A, H, W)
    inw_out = inw.reshape(B, 4 * A, H, W)
    outw_out = outw.reshape(B, 4 * A, H, W)
    return labels_out, targets_out, inw_out, outw_out
